# baseline (speedup 1.0000x reference)
import functools

import jax
import jax.numpy as jnp
from jax import lax
from jax.experimental import pallas as pl
from jax.experimental.pallas import tpu as pltpu
from jax.experimental.pallas import tpu_sc as plsc

BATCH = 4
SEQ = 2048
DIM = 1024
LANES = 16
NUM_CORES = 2
NUM_SUBCORES = 16
NW = NUM_CORES * NUM_SUBCORES
SEQ_PER_W = SEQ // NW
XCHUNK = 32
VECS_PER_ROW = DIM // LANES


def _sc_add_kernel(x_hbm, pos_hbm, out_hbm, pos_v, x_v, sem):
    wid = lax.axis_index("s") * NUM_CORES + lax.axis_index("c")
    base = wid * SEQ_PER_W
    pltpu.sync_copy(pos_hbm.at[pl.ds(base, SEQ_PER_W)], pos_v)
    for b in range(BATCH):
        for c in range(SEQ_PER_W // XCHUNK):
            row0 = b * SEQ + base + c * XCHUNK
            pltpu.sync_copy(x_hbm.at[pl.ds(row0, XCHUNK)], x_v)

            def row_body(r, carry, coff=c * XCHUNK):
                for j in range(VECS_PER_ROW):
                    sl = pl.ds(j * LANES, LANES)
                    x_v[r, sl] = x_v[r, sl] + pos_v[coff + r, sl]
                return carry

            lax.fori_loop(0, XCHUNK, row_body, 0)
            pltpu.sync_copy(x_v, out_hbm.at[pl.ds(row0, XCHUNK)])


def kernel(x, pos_table):
    batch, seq_len, dim = x.shape
    x2 = x.reshape(batch * seq_len, dim)
    run = functools.partial(
        pl.kernel,
        mesh=plsc.VectorSubcoreMesh(core_axis_name="c", subcore_axis_name="s"),
        out_type=jax.ShapeDtypeStruct((batch * seq_len, dim), x.dtype),
        scratch_types=[
            pltpu.VMEM((SEQ_PER_W, DIM), jnp.float32),
            pltpu.VMEM((XCHUNK, DIM), jnp.float32),
            pltpu.SemaphoreType.DMA,
        ],
    )(_sc_add_kernel)
    out2 = run(x2, pos_table[:seq_len])
    return out2.reshape(batch, seq_len, dim)


# speedup vs baseline: 1.2331x; 1.2331x over previous
import functools

import jax
import jax.numpy as jnp
from jax import lax
from jax.experimental import pallas as pl
from jax.experimental.pallas import tpu as pltpu
from jax.experimental.pallas import tpu_sc as plsc

BATCH = 4
SEQ = 2048
DIM = 1024
LANES = 16
NUM_CORES = 2
NUM_SUBCORES = 16
NW = NUM_CORES * NUM_SUBCORES
SEQ_PER_W = SEQ // NW
POS_PASS = 32
XCHUNK = 16
NBUF = 2
CHUNKS_PER_PASS = BATCH * (POS_PASS // XCHUNK)
N_OUTER = CHUNKS_PER_PASS // NBUF
VECS_PER_ROW = DIM // LANES


def _sc_add_kernel(x_hbm, pos_hbm, out_hbm, pos_v,
                   ib0, ib1, ob0, ob1, is0, is1, os0, os1):
    in_bufs = [ib0, ib1]
    out_bufs = [ob0, ob1]
    in_sems = [is0, is1]
    out_sems = [os0, os1]
    wid = lax.axis_index("s") * NUM_CORES + lax.axis_index("c")
    base = wid * SEQ_PER_W

    for p in range(SEQ_PER_W // POS_PASS):
        pbase = base + p * POS_PASS
        pltpu.sync_copy(pos_hbm.at[pl.ds(pbase, POS_PASS)], pos_v)

        def row0_of(t):
            b = t // (POS_PASS // XCHUNK)
            c = t % (POS_PASS // XCHUNK)
            return b * SEQ + pbase + c * XCHUNK

        for i in range(NBUF):
            pltpu.make_async_copy(
                x_hbm.at[pl.ds(row0_of(i), XCHUNK)], in_bufs[i], in_sems[i]
            ).start()

        def outer(o, carry):
            for i in range(NBUF):
                t = o * NBUF + i
                r0 = row0_of(t)
                pltpu.make_async_copy(
                    x_hbm.at[pl.ds(r0, XCHUNK)], in_bufs[i], in_sems[i]
                ).wait()

                @pl.when(t >= NBUF)
                def _():
                    pltpu.make_async_copy(
                        out_bufs[i], out_hbm.at[pl.ds(r0, XCHUNK)], out_sems[i]
                    ).wait()

                coff = (t % (POS_PASS // XCHUNK)) * XCHUNK

                def row_body(r, rc, i=i, coff=coff):
                    for j in range(VECS_PER_ROW):
                        sl = pl.ds(j * LANES, LANES)
                        out_bufs[i][r, sl] = in_bufs[i][r, sl] + pos_v[coff + r, sl]
                    return rc

                lax.fori_loop(0, XCHUNK, row_body, 0)
                pltpu.make_async_copy(
                    out_bufs[i], out_hbm.at[pl.ds(r0, XCHUNK)], out_sems[i]
                ).start()

                nt = t + NBUF

                @pl.when(nt < CHUNKS_PER_PASS)
                def _():
                    pltpu.make_async_copy(
                        x_hbm.at[pl.ds(row0_of(nt), XCHUNK)], in_bufs[i], in_sems[i]
                    ).start()

            return carry

        lax.fori_loop(0, N_OUTER, outer, 0)

        for i in range(NBUF):
            t = CHUNKS_PER_PASS - NBUF + i
            pltpu.make_async_copy(
                out_bufs[i], out_hbm.at[pl.ds(row0_of(t), XCHUNK)], out_sems[i]
            ).wait()


def kernel(x, pos_table):
    batch, seq_len, dim = x.shape
    x2 = x.reshape(batch * seq_len, dim)
    run = functools.partial(
        pl.kernel,
        mesh=plsc.VectorSubcoreMesh(core_axis_name="c", subcore_axis_name="s"),
        out_type=jax.ShapeDtypeStruct((batch * seq_len, dim), x.dtype),
        scratch_types=[
            pltpu.VMEM((POS_PASS, DIM), jnp.float32),
            pltpu.VMEM((XCHUNK, DIM), jnp.float32),
            pltpu.VMEM((XCHUNK, DIM), jnp.float32),
            pltpu.VMEM((XCHUNK, DIM), jnp.float32),
            pltpu.VMEM((XCHUNK, DIM), jnp.float32),
            pltpu.SemaphoreType.DMA,
            pltpu.SemaphoreType.DMA,
            pltpu.SemaphoreType.DMA,
            pltpu.SemaphoreType.DMA,
        ],
    )(_sc_add_kernel)
    out2 = run(x2, pos_table[:seq_len])
    return out2.reshape(batch, seq_len, dim)


# speedup vs baseline: 3.3525x; 2.7188x over previous
import jax
import jax.numpy as jnp
from jax.experimental import pallas as pl


def _add_pos_block(x_ref, pos_ref, o_ref):
    o_ref[...] = x_ref[...] + pos_ref[...][None, :, :]


def kernel(x, pos_table):
    batch, seq_len, dim = x.shape
    block_s = 512
    grid = (seq_len // block_s,)
    return pl.pallas_call(
        _add_pos_block,
        grid=grid,
        in_specs=[
            pl.BlockSpec((batch, block_s, dim), lambda i: (0, i, 0)),
            pl.BlockSpec((block_s, dim), lambda i: (i, 0)),
        ],
        out_specs=pl.BlockSpec((batch, block_s, dim), lambda i: (0, i, 0)),
        out_shape=jax.ShapeDtypeStruct((batch, seq_len, dim), x.dtype),
    )(x, pos_table[:seq_len])
